# baseline (speedup 1.0000x reference)
import functools

import jax
import jax.numpy as jnp
from jax import lax
from jax.experimental import pallas as pl
from jax.experimental.pallas import tpu as pltpu
from jax.experimental.pallas import tpu_sc as plsc

EMB = 64
BATCH = 16384
PK = 2 * EMB
NPACK = 500000
NC = 2
NS = 16
NW = NC * NS
BPW = BATCH // NW
IDXW = 128
CH = BPW // IDXW

_sc_mesh = plsc.VectorSubcoreMesh(core_axis_name="c", subcore_axis_name="s")


@functools.partial(
    pl.kernel,
    out_type=[
        jax.ShapeDtypeStruct((BATCH, PK), jnp.float32),
        jax.ShapeDtypeStruct((BATCH, PK), jnp.float32),
    ],
    mesh=_sc_mesh,
    scratch_types=[
        pltpu.VMEM((CH, IDXW), jnp.int32),
        pltpu.VMEM((BPW, PK), jnp.float32),
        pltpu.SemaphoreType.DMA,
    ],
)
def _sc_gather(uidx_hbm, iidx_hbm, utab_hbm, itab_hbm, urows_hbm, irows_hbm,
               idx_v, rows_v, sem):
    wid = lax.axis_index("s") * NC + lax.axis_index("c")
    base = wid * BPW

    def one_table(idx_hbm, tab_hbm, out_hbm):
        pltpu.sync_copy(idx_hbm.at[wid], idx_v)
        handles = []
        for j in range(CH):
            handles.append(pltpu.async_copy(
                tab_hbm.at[idx_v.at[j]],
                rows_v.at[pl.ds(j * IDXW, IDXW)], sem))
        for h in handles:
            h.wait()
        pltpu.sync_copy(rows_v, out_hbm.at[pl.ds(base, BPW)])

    one_table(uidx_hbm, utab_hbm, urows_hbm)
    one_table(iidx_hbm, itab_hbm, irows_hbm)


ROWS = 1024
GRID = BATCH // ROWS


def _mlp_body(u_ref, i_ref, uodd_ref, iodd_ref, w1a_hi_ref, w1a_lo_ref,
              w1b_hi_ref, w1b_lo_ref, b1_ref, w2_ref, b2_ref, w3_ref, b3_ref,
              out_ref):
    u = u_ref[...]
    i = i_ref[...]
    hu_hi = jnp.dot(u, w1a_hi_ref[...], preferred_element_type=jnp.float32)
    hu_lo = jnp.dot(u, w1a_lo_ref[...], preferred_element_type=jnp.float32)
    hu = jnp.where(uodd_ref[...] > 0.5, hu_lo, hu_hi)
    hi_hi = jnp.dot(i, w1b_hi_ref[...], preferred_element_type=jnp.float32)
    hi_lo = jnp.dot(i, w1b_lo_ref[...], preferred_element_type=jnp.float32)
    hi = jnp.where(iodd_ref[...] > 0.5, hi_lo, hi_hi)
    h = jnp.maximum(hu + hi + b1_ref[...], 0.0)
    h = jnp.maximum(
        jnp.dot(h, w2_ref[...], preferred_element_type=jnp.float32) + b2_ref[...],
        0.0)
    out_ref[...] = (
        jnp.dot(h, w3_ref[...], preferred_element_type=jnp.float32) + b3_ref[...])


_tc_mlp = pl.pallas_call(
    _mlp_body,
    grid=(GRID,),
    in_specs=[
        pl.BlockSpec((ROWS, PK), lambda i: (i, 0)),
        pl.BlockSpec((ROWS, PK), lambda i: (i, 0)),
        pl.BlockSpec((ROWS, 1), lambda i: (i, 0)),
        pl.BlockSpec((ROWS, 1), lambda i: (i, 0)),
        pl.BlockSpec((PK, 64), lambda i: (0, 0)),
        pl.BlockSpec((PK, 64), lambda i: (0, 0)),
        pl.BlockSpec((PK, 64), lambda i: (0, 0)),
        pl.BlockSpec((PK, 64), lambda i: (0, 0)),
        pl.BlockSpec((1, 64), lambda i: (0, 0)),
        pl.BlockSpec((64, 32), lambda i: (0, 0)),
        pl.BlockSpec((1, 32), lambda i: (0, 0)),
        pl.BlockSpec((32, 1), lambda i: (0, 0)),
        pl.BlockSpec((1, 1), lambda i: (0, 0)),
    ],
    out_specs=pl.BlockSpec((ROWS, 1), lambda i: (i, 0)),
    out_shape=jax.ShapeDtypeStruct((BATCH, 1), jnp.float32),
)


def kernel(user, item, user_table, item_table, W1, b1, W2, b2, W3, b3):
    user = user.astype(jnp.int32)
    item = item.astype(jnp.int32)
    upk = user_table.reshape(NPACK, PK)
    ipk = item_table.reshape(NPACK, PK)
    ug = (user >> 1).reshape(NW, CH, IDXW)
    ig = (item >> 1).reshape(NW, CH, IDXW)
    uodd = (user & 1).astype(jnp.float32).reshape(BATCH, 1)
    iodd = (item & 1).astype(jnp.float32).reshape(BATCH, 1)
    urows, irows = _sc_gather(ug, ig, upk, ipk)
    W1a, W1b = W1[:EMB], W1[EMB:]
    z = jnp.zeros((EMB, 64), jnp.float32)
    w1a_hi = jnp.concatenate([W1a, z], axis=0)
    w1a_lo = jnp.concatenate([z, W1a], axis=0)
    w1b_hi = jnp.concatenate([W1b, z], axis=0)
    w1b_lo = jnp.concatenate([z, W1b], axis=0)
    out = _tc_mlp(urows, irows, uodd, iodd, w1a_hi, w1a_lo, w1b_hi, w1b_lo,
                  b1.reshape(1, EMB), W2, b2.reshape(1, 32), W3,
                  b3.reshape(1, 1))
    return out.reshape(BATCH)


# speedup vs baseline: 1.0016x; 1.0016x over previous
import functools

import jax
import jax.numpy as jnp
from jax import lax
from jax.experimental import pallas as pl
from jax.experimental.pallas import tpu as pltpu
from jax.experimental.pallas import tpu_sc as plsc

EMB = 64
BATCH = 16384
NC = 2
NS = 16
NW = NC * NS
BPW = BATCH // NW
PAIR_ROWS = 500000

_sc_mesh = plsc.VectorSubcoreMesh(core_axis_name="c", subcore_axis_name="s")


@functools.partial(
    pl.kernel,
    out_type=[
        jax.ShapeDtypeStruct((BATCH, 128), jnp.float32),
        jax.ShapeDtypeStruct((BATCH, 128), jnp.float32),
    ],
    mesh=_sc_mesh,
    scratch_types=[
        pltpu.VMEM((BPW,), jnp.int32),
        pltpu.VMEM((BPW, 128), jnp.float32),
        pltpu.SemaphoreType.DMA,
    ],
)
def _sc_gather(urow_hbm, irow_hbm, utab_hbm, itab_hbm, gu_hbm, gi_hbm,
               idx_v, rows_v, sem):
    wid = lax.axis_index("s") * NC + lax.axis_index("c")
    base = wid * BPW

    pltpu.sync_copy(urow_hbm.at[pl.ds(base, BPW)], idx_v)
    pltpu.async_copy(utab_hbm.at[idx_v], rows_v, sem).wait()
    pltpu.sync_copy(rows_v, gu_hbm.at[pl.ds(base, BPW)])

    pltpu.sync_copy(irow_hbm.at[pl.ds(base, BPW)], idx_v)
    pltpu.async_copy(itab_hbm.at[idx_v], rows_v, sem).wait()
    pltpu.sync_copy(rows_v, gi_hbm.at[pl.ds(base, BPW)])


NB = 2048
GRID = BATCH // NB


def _mlp_body(gu_ref, gi_ref, pu_ref, pi_ref, w1a_ref, w1b_ref, b1_ref,
              w2_ref, b2_ref, w3_ref, b3_ref, out_ref):
    gu = gu_ref[...]
    gi = gi_ref[...]
    xu = jnp.where(pu_ref[...] > 0.5, gu[:, EMB:], gu[:, :EMB])
    xi = jnp.where(pi_ref[...] > 0.5, gi[:, EMB:], gi[:, :EMB])
    h = jnp.dot(xu, w1a_ref[...], preferred_element_type=jnp.float32)
    h = h + jnp.dot(xi, w1b_ref[...], preferred_element_type=jnp.float32)
    h = jnp.maximum(h + b1_ref[...], 0.0)
    h = jnp.maximum(
        jnp.dot(h, w2_ref[...], preferred_element_type=jnp.float32)
        + b2_ref[...], 0.0)
    out_ref[...] = (
        jnp.dot(h, w3_ref[...], preferred_element_type=jnp.float32)
        + b3_ref[...])


_tc_mlp = pl.pallas_call(
    _mlp_body,
    grid=(GRID,),
    in_specs=[
        pl.BlockSpec((NB, 128), lambda i: (i, 0)),
        pl.BlockSpec((NB, 128), lambda i: (i, 0)),
        pl.BlockSpec((NB, 1), lambda i: (i, 0)),
        pl.BlockSpec((NB, 1), lambda i: (i, 0)),
        pl.BlockSpec((EMB, EMB), lambda i: (0, 0)),
        pl.BlockSpec((EMB, EMB), lambda i: (0, 0)),
        pl.BlockSpec((1, EMB), lambda i: (0, 0)),
        pl.BlockSpec((EMB, 32), lambda i: (0, 0)),
        pl.BlockSpec((1, 32), lambda i: (0, 0)),
        pl.BlockSpec((32, 1), lambda i: (0, 0)),
        pl.BlockSpec((1, 1), lambda i: (0, 0)),
    ],
    out_specs=pl.BlockSpec((NB, 1), lambda i: (i, 0)),
    out_shape=jax.ShapeDtypeStruct((BATCH, 1), jnp.float32),
)


def kernel(user, item, user_table, item_table, W1, b1, W2, b2, W3, b3):
    user = user.astype(jnp.int32)
    item = item.astype(jnp.int32)
    pu = (user & 1).astype(jnp.float32).reshape(BATCH, 1)
    pv = (item & 1).astype(jnp.float32).reshape(BATCH, 1)
    urow = lax.shift_right_logical(user, 1)
    irow = lax.shift_right_logical(item, 1)
    tab_u = user_table.reshape(PAIR_ROWS, 128)
    tab_i = item_table.reshape(PAIR_ROWS, 128)
    gu, gi = _sc_gather(urow, irow, tab_u, tab_i)
    out = _tc_mlp(gu, gi, pu, pv, W1[:EMB], W1[EMB:], b1.reshape(1, EMB),
                  W2, b2.reshape(1, 32), W3, b3.reshape(1, 1))
    return out.reshape(BATCH)
